# Initial kernel scaffold; baseline (speedup 1.0000x reference)
import jax
import jax.numpy as jnp
from jax.experimental import pallas as pl


def kernel(weight, indices):
    raise NotImplementedError("write your pallas kernel here")


# speedup vs baseline: 1.1085x; 1.1085x over previous
import functools

import jax
import jax.numpy as jnp
from jax import lax
from jax.experimental import pallas as pl
from jax.experimental.pallas import tpu as pltpu
from jax.experimental.pallas import tpu_sc as plsc

NUM_EMB = 1000000
DIM = 32
BATCH = 16384
HIST = 50
B = BATCH * HIST

NC, NS = 2, 16
NW = NC * NS
B_PER_W = B // NW
SLICE = 128
K = 20
CHUNK = SLICE * K
G = B_PER_W // CHUNK


def _emb_body(table_hbm, idx_hbm, out_hbm, idx_v, rows_v, sem):
    wid = lax.axis_index("s") * NC + lax.axis_index("c")
    base = wid * B_PER_W

    @pl.loop(0, G)
    def _chunk(g):
        start = base + g * CHUNK
        pltpu.sync_copy(idx_hbm.at[pl.ds(start, CHUNK)], idx_v)
        cps = [
            pltpu.async_copy(
                table_hbm.at[idx_v.at[pl.ds(j * SLICE, SLICE)]],
                rows_v.at[pl.ds(j * SLICE, SLICE)],
                sem,
            )
            for j in range(K)
        ]
        for cp in cps:
            cp.wait()
        pltpu.sync_copy(rows_v, out_hbm.at[pl.ds(start, CHUNK)])


@jax.jit
def _emb_lookup(weight, idx_flat):
    mesh = plsc.VectorSubcoreMesh(core_axis_name="c", subcore_axis_name="s")
    return pl.kernel(
        _emb_body,
        out_type=jax.ShapeDtypeStruct((B, DIM), jnp.float32),
        mesh=mesh,
        scratch_types=[
            pltpu.VMEM((CHUNK,), jnp.int32),
            pltpu.VMEM((CHUNK, DIM), jnp.float32),
            pltpu.SemaphoreType.DMA,
        ],
        compiler_params=pltpu.CompilerParams(use_tc_tiling_on_sc=False),
    )(weight, idx_flat)


def kernel(weight, indices):
    idx_flat = indices.reshape(-1).astype(jnp.int32)
    out = _emb_lookup(weight, idx_flat)
    return out.reshape(BATCH, HIST, DIM)


# speedup vs baseline: 1.1087x; 1.0001x over previous
import functools

import jax
import jax.numpy as jnp
from jax import lax
from jax.experimental import pallas as pl
from jax.experimental.pallas import tpu as pltpu
from jax.experimental.pallas import tpu_sc as plsc

NUM_EMB = 1000000
DIM = 32
BATCH = 16384
HIST = 50
B = BATCH * HIST

NC, NS = 2, 16
NW = NC * NS
B_PER_W = B // NW
SLICE = 128
K = 10
CHUNK = SLICE * K
G = B_PER_W // CHUNK


def _fire_gathers(table_hbm, idx_v, rows_v, sem):
    return [
        pltpu.async_copy(
            table_hbm.at[idx_v.at[pl.ds(j * SLICE, SLICE)]],
            rows_v.at[pl.ds(j * SLICE, SLICE)],
            sem,
        )
        for j in range(K)
    ]


def _emb_body(table_hbm, idx_hbm, out_hbm, idx0, idx1, rows0, rows1, sem0, sem1):
    wid = lax.axis_index("s") * NC + lax.axis_index("c")
    base = wid * B_PER_W
    idx_v = (idx0, idx1)
    rows_v = (rows0, rows1)
    sems = (sem0, sem1)

    pltpu.sync_copy(idx_hbm.at[pl.ds(base, CHUNK)], idx0)
    _fire_gathers(table_hbm, idx0, rows0, sem0)

    @pl.loop(0, G, step=2)
    def _chunk(g):
        for b in range(2):
            gg = g + b
            nb = 1 - b
            @pl.when(gg + 1 < G)
            def _prefetch():
                pltpu.sync_copy(
                    idx_hbm.at[pl.ds(base + (gg + 1) * CHUNK, CHUNK)], idx_v[nb]
                )
                _fire_gathers(table_hbm, idx_v[nb], rows_v[nb], sems[nb])

            for j in range(K):
                pltpu.make_async_copy(
                    table_hbm.at[idx_v[b].at[pl.ds(j * SLICE, SLICE)]],
                    rows_v[b].at[pl.ds(j * SLICE, SLICE)],
                    sems[b],
                ).wait()
            pltpu.sync_copy(rows_v[b], out_hbm.at[pl.ds(base + gg * CHUNK, CHUNK)])


@jax.jit
def _emb_lookup(weight, idx_flat):
    mesh = plsc.VectorSubcoreMesh(core_axis_name="c", subcore_axis_name="s")
    return pl.kernel(
        _emb_body,
        out_type=jax.ShapeDtypeStruct((B, DIM), jnp.float32),
        mesh=mesh,
        scratch_types=[
            pltpu.VMEM((CHUNK,), jnp.int32),
            pltpu.VMEM((CHUNK,), jnp.int32),
            pltpu.VMEM((CHUNK, DIM), jnp.float32),
            pltpu.VMEM((CHUNK, DIM), jnp.float32),
            pltpu.SemaphoreType.DMA,
            pltpu.SemaphoreType.DMA,
        ],
        compiler_params=pltpu.CompilerParams(use_tc_tiling_on_sc=False),
    )(weight, idx_flat)


def kernel(weight, indices):
    idx_flat = indices.reshape(-1).astype(jnp.int32)
    out = _emb_lookup(weight, idx_flat)
    return out.reshape(BATCH, HIST, DIM)


# speedup vs baseline: 1.7957x; 1.6197x over previous
import functools

import jax
import jax.numpy as jnp
from jax import lax
from jax.experimental import pallas as pl
from jax.experimental.pallas import tpu as pltpu
from jax.experimental.pallas import tpu_sc as plsc

NUM_EMB = 1000000
DIM = 32
BATCH = 16384
HIST = 50
B = BATCH * HIST

NC, NS = 2, 16
NW = NC * NS
B_PER_W = B // NW
SLICE = 128
K = 10
CHUNK = SLICE * K
G = B_PER_W // CHUNK


def _fire_gathers(table_hbm, idx_v, rows_v, sem):
    return [
        pltpu.async_copy(
            table_hbm.at[idx_v.at[pl.ds(j * SLICE, SLICE)]],
            rows_v.at[pl.ds(j * SLICE, SLICE)],
            sem,
        )
        for j in range(K)
    ]


def _emb_body(table_hbm, idx_hbm, out_hbm, idx0, idx1, rows0, rows1, sem0, sem1):
    wid = lax.axis_index("s") * NC + lax.axis_index("c")
    base = wid * B_PER_W
    idx_v = (idx0, idx1)
    rows_v = (rows0, rows1)
    sems = (sem0, sem1)

    pltpu.sync_copy(idx_hbm.at[pl.ds(base, CHUNK)], idx0)
    _fire_gathers(table_hbm, idx0, rows0, sem0)

    @pl.loop(0, G, step=2)
    def _chunk(g):
        for b in range(2):
            gg = g + b
            nb = 1 - b
            @pl.when(gg + 1 < G)
            def _prefetch():
                pltpu.sync_copy(
                    idx_hbm.at[pl.ds(base + (gg + 1) * CHUNK, CHUNK)], idx_v[nb]
                )
                _fire_gathers(table_hbm, idx_v[nb], rows_v[nb], sems[nb])

            for j in range(K):
                pltpu.make_async_copy(
                    table_hbm.at[idx_v[b].at[pl.ds(j * SLICE, SLICE)]],
                    rows_v[b].at[pl.ds(j * SLICE, SLICE)],
                    sems[b],
                ).wait()
            pltpu.sync_copy(rows_v[b], out_hbm.at[pl.ds(base + gg * CHUNK, CHUNK)])


@jax.jit
def _emb_lookup(weight, idx_flat):
    mesh = plsc.VectorSubcoreMesh(core_axis_name="c", subcore_axis_name="s")
    return pl.kernel(
        _emb_body,
        out_type=jax.ShapeDtypeStruct((B, DIM), jnp.float32),
        mesh=mesh,
        scratch_types=[
            pltpu.VMEM((CHUNK,), jnp.int32),
            pltpu.VMEM((CHUNK,), jnp.int32),
            pltpu.VMEM((CHUNK, DIM), jnp.float32),
            pltpu.VMEM((CHUNK, DIM), jnp.float32),
            pltpu.SemaphoreType.DMA,
            pltpu.SemaphoreType.DMA,
        ],
        compiler_params=pltpu.CompilerParams(use_tc_tiling_on_sc=False),
    )(weight, idx_flat)


def kernel(weight, indices):
    idx_flat = indices.reshape(-1).astype(jnp.int32)
    wlin = jax.lax.optimization_barrier(weight.reshape(NUM_EMB // 4, DIM * 4))
    table = wlin.reshape(NUM_EMB, DIM)
    out = _emb_lookup(table, idx_flat)
    outp = jax.lax.optimization_barrier(out.reshape(B // 4, DIM * 4))
    return outp.reshape(BATCH, HIST, DIM)
